# baseline (speedup 1.0000x reference)
import jax
import jax.numpy as jnp
from jax.experimental import pallas as pl
from jax.experimental.pallas import tpu as pltpu

EPS = 0.001
BS = 512


def _norm_kernel(ids_ref, x_ref, mean_ref, std_ref, o_ref):
    scale = 1.0 / (jnp.exp(std_ref[...]) + EPS)
    o_ref[...] = (x_ref[...] - mean_ref[...]) * scale


def kernel(x, context_id, initial_mean, initial_std):
    B, S, D = x.shape
    C = initial_mean.shape[0]
    ids = context_id.reshape(-1)
    mean3 = initial_mean.reshape(C, 1, D)
    std3 = initial_std.reshape(C, 1, D)
    grid = (B, S // BS)
    grid_spec = pltpu.PrefetchScalarGridSpec(
        num_scalar_prefetch=1,
        grid=grid,
        in_specs=[
            pl.BlockSpec((1, BS, D), lambda b, s, ids: (b, s, 0)),
            pl.BlockSpec((1, 1, D), lambda b, s, ids: (ids[b], 0, 0)),
            pl.BlockSpec((1, 1, D), lambda b, s, ids: (ids[b], 0, 0)),
        ],
        out_specs=pl.BlockSpec((1, BS, D), lambda b, s, ids: (b, s, 0)),
    )
    return pl.pallas_call(
        _norm_kernel,
        grid_spec=grid_spec,
        out_shape=jax.ShapeDtypeStruct((B, S, D), x.dtype),
    )(ids, x, mean3, std3)


# speedup vs baseline: 1.0895x; 1.0895x over previous
import jax
import jax.numpy as jnp
from jax.experimental import pallas as pl
from jax.experimental.pallas import tpu as pltpu

EPS = 0.001
BS = 1024


def _norm_kernel(ids_ref, x_ref, mean_ref, std_ref, o_ref):
    scale = 1.0 / (jnp.exp(std_ref[0]) + EPS)
    o_ref[...] = (x_ref[...] - mean_ref[0]) * scale


def kernel(x, context_id, initial_mean, initial_std):
    B, S, D = x.shape
    C = initial_mean.shape[0]
    nblk_per_batch = S // BS
    ids = context_id.reshape(-1)
    x2 = x.reshape(B * S, D)
    mean3 = initial_mean.reshape(C, 1, D)
    std3 = initial_std.reshape(C, 1, D)
    grid = (B * S // BS,)
    grid_spec = pltpu.PrefetchScalarGridSpec(
        num_scalar_prefetch=1,
        grid=grid,
        in_specs=[
            pl.BlockSpec((BS, D), lambda i, ids: (i, 0)),
            pl.BlockSpec((1, 1, D), lambda i, ids: (ids[i // nblk_per_batch], 0, 0)),
            pl.BlockSpec((1, 1, D), lambda i, ids: (ids[i // nblk_per_batch], 0, 0)),
        ],
        out_specs=pl.BlockSpec((BS, D), lambda i, ids: (i, 0)),
    )
    out = pl.pallas_call(
        _norm_kernel,
        grid_spec=grid_spec,
        out_shape=jax.ShapeDtypeStruct((B * S, D), x.dtype),
        compiler_params=pltpu.CompilerParams(
            dimension_semantics=("arbitrary",),
        ),
    )(ids, x2, mean3, std3)
    return out.reshape(B, S, D)


# speedup vs baseline: 1.1651x; 1.0693x over previous
import jax
import jax.numpy as jnp
from jax.experimental import pallas as pl
from jax.experimental.pallas import tpu as pltpu

EPS = 0.001
BS = 2048


def _norm_kernel(ids_ref, x_ref, mean_ref, std_ref, o_ref):
    scale = 1.0 / (jnp.exp(std_ref[0]) + EPS)
    o_ref[...] = (x_ref[...] - mean_ref[0]) * scale


def kernel(x, context_id, initial_mean, initial_std):
    B, S, D = x.shape
    C = initial_mean.shape[0]
    nblk_per_batch = S // BS
    ids = context_id.reshape(-1)
    x2 = x.reshape(B * S, D)
    mean3 = initial_mean.reshape(C, 1, D)
    std3 = initial_std.reshape(C, 1, D)
    grid = (B * S // BS,)
    grid_spec = pltpu.PrefetchScalarGridSpec(
        num_scalar_prefetch=1,
        grid=grid,
        in_specs=[
            pl.BlockSpec((BS, D), lambda i, ids: (i, 0)),
            pl.BlockSpec((1, 1, D), lambda i, ids: (ids[i // nblk_per_batch], 0, 0)),
            pl.BlockSpec((1, 1, D), lambda i, ids: (ids[i // nblk_per_batch], 0, 0)),
        ],
        out_specs=pl.BlockSpec((BS, D), lambda i, ids: (i, 0)),
    )
    out = pl.pallas_call(
        _norm_kernel,
        grid_spec=grid_spec,
        out_shape=jax.ShapeDtypeStruct((B * S, D), x.dtype),
        compiler_params=pltpu.CompilerParams(
            dimension_semantics=("arbitrary",),
        ),
    )(ids, x2, mean3, std3)
    return out.reshape(B, S, D)
